# baseline (speedup 1.0000x reference)
import functools

import jax
import jax.numpy as jnp
from jax import lax
from jax.experimental import pallas as pl
from jax.experimental.pallas import tpu as pltpu
from jax.experimental.pallas import tpu_sc as plsc

K = 64
N = 32768
ROWS = 128
NVEC = N // 16
GT_BUF = 256
EQ_BUF = 80
GT_CAP = 240
HIST_EXIT = 96
KSAMP = 32

_U = jnp.uint32


def _keys(v):
  kb = plsc.bitcast(v, jnp.uint32)
  sign = kb >> _U(31)
  flip = (_U(0) - sign) | _U(0x80000000)
  return kb ^ flip


def _scalar(x):
  return x[0]


def _sc_body(in_hbm, out_hbm, row_a, row_b, hist_v, gt_k, gt_i, eq_i, outst,
             sem_a, sem_b):
  c = lax.axis_index("c")
  s = lax.axis_index("s")
  wid = c * 16 + s
  iota = lax.iota(jnp.int32, 16)
  ones = jnp.ones((16,), jnp.int32)
  zeros_u = jnp.zeros((16,), jnp.uint32)
  zeros_i = jnp.zeros((16,), jnp.int32)

  def clear_hist(nwords=4096):
    @plsc.parallel_loop(0, nwords, step=16, unroll=8)
    def _hc(i):
      hist_v[pl.ds(i, 16)] = zeros_i

  def clear_gt():
    for ci in range(GT_BUF // 16):
      gt_k[pl.ds(ci * 16, 16)] = zeros_u

  def scan_hist(base, nb_vec, kneed, acc0):
    static = isinstance(nb_vec, int)
    init = (acc0, jnp.int32(0), acc0, jnp.bool_(False))

    @pl.loop(0, nb_vec, init_carry=init, unroll=8 if static and nb_vec >= 8
             else None)
    def scan_res(i, sc):
      acc, vvi, vacc, found = sc
      vi = nb_vec - 1 - i
      cum = jnp.cumsum(hist_v[pl.ds(base + vi * 16, 16)])
      acc2 = acc + cum[15]
      crossed = jnp.logical_and(acc2 >= kneed, jnp.logical_not(found))
      vvi = jnp.where(crossed, vi, vvi)
      vacc = jnp.where(crossed, acc, vacc)
      return (acc2, vvi, vacc, found | crossed)

    _, vvi, vacc, _ = scan_res
    vec = hist_v[pl.ds(base + vvi * 16, 16)]
    cum = jnp.cumsum(vec)
    suf = vacc + cum[15] - cum + vec
    m = suf >= kneed
    blane = jnp.max(jnp.where(m, iota, -1))
    sel = iota == blane
    SB = jnp.max(jnp.where(sel, suf, 0))
    histB = jnp.max(jnp.where(sel, vec, 0))
    return vvi * 16 + blane, SB, histB

  def collect_gt(row_v, T):
    @plsc.parallel_loop(0, N, step=16, unroll=16,
                        carry=jnp.zeros((16,), jnp.int32))
    def cnts(i, cgv):
      ku = _keys(row_v[pl.ds(i, 16)])
      mgt = ku > T
      cs = jnp.minimum(_scalar(cgv), GT_CAP)
      plsc.store_compressed(gt_k.at[pl.ds(cs, 16)], ku, mask=mgt)
      plsc.store_compressed(gt_i.at[pl.ds(cs, 16)], iota + i, mask=mgt)
      return cgv + plsc.all_reduce_population_count(mgt)

    return _scalar(cnts)

  def process(row_v, row):
    clear_gt()
    clear_hist(4352)

    @plsc.parallel_loop(0, N, step=64, unroll=8)
    def _samp(i):
      ku = _keys(row_v[pl.ds(i, 16)])
      b = (ku >> _U(20)).astype(jnp.int32)
      plsc.addupdate_scatter(hist_v, [b], ones)
      plsc.addupdate_scatter(hist_v, [4096 + (b >> 4)], ones)

    ks = jnp.int32(KSAMP)
    G, SG, hG = scan_hist(4096, 16, ks, jnp.int32(0))
    Br, _, _ = scan_hist(G * 16, 1, ks, SG - hG)
    T = (G * 16 + Br).astype(jnp.uint32) << _U(20)

    ct = collect_gt(row_v, T)

    fb = jnp.logical_or(ct < K, ct > GT_CAP)

    @pl.loop(0, jnp.where(fb, 1, 0), init_carry=(ct, jnp.int32(0)))
    def cg_ce(_t, _c):
      clear_hist()

      @plsc.parallel_loop(0, N, step=16, unroll=8)
      def _full(i):
        ku = _keys(row_v[pl.ds(i, 16)])
        b = (ku >> _U(20)).astype(jnp.int32)
        plsc.addupdate_scatter(hist_v, [b], ones)

      B, SB, histB = scan_hist(0, 256, jnp.int32(K), jnp.int32(0))
      T0 = B.astype(jnp.uint32) << _U(20)

      refine = histB > HIST_EXIT
      init = (_U(0xFFF00000), T0, jnp.int32(K) - (SB - histB), T0)

      @pl.loop(0, jnp.where(refine, 2, 0), init_carry=init)
      def ref_res(l, cr):
        pmask, pval, kneed, _ = cr
        is1 = l == 0
        shift = jnp.where(is1, _U(8), _U(0))
        bmask = jnp.where(is1, _U(0xFFF), _U(0xFF))
        nb_vec = jnp.where(is1, 256, 16)

        @pl.loop(0, nb_vec)
        def _c(ci):
          hist_v[pl.ds(ci * 16, 16)] = zeros_i

        def hp(i, _):
          ku = _keys(row_v[pl.ds(i * 16, 16)])
          part = (ku & pmask) == pval
          b = ((ku >> shift) & bmask).astype(jnp.int32)
          plsc.addupdate_scatter(hist_v, [b], ones, mask=part)
          return 0
        lax.fori_loop(0, NVEC, hp, 0)

        B2, SB2, histB2 = scan_hist(0, nb_vec, kneed, jnp.int32(0))
        T2 = pval | (B2.astype(jnp.uint32) << shift)
        return (pmask | (bmask << shift), T2, kneed - (SB2 - histB2), T2)

      Tx = ref_res[3]
      clear_gt()
      cgx = collect_gt(row_v, Tx)

      @pl.loop(0, jnp.where(cgx < K, 1, 0), init_carry=jnp.int32(0))
      def cex(_t2, _ce):
        def eqp(i, cev):
          ku = _keys(row_v[pl.ds(i * 16, 16)])
          meq = jnp.logical_and(ku == Tx, cev < K)
          plsc.store_compressed(eq_i.at[pl.ds(_scalar(cev), 16)],
                                iota + i * 16, mask=meq)
          return cev + plsc.all_reduce_population_count(meq)
        return _scalar(lax.fori_loop(0, NVEC, eqp, zeros_i))

      for a in range(EQ_BUF // 16):
        lanepos = iota + a * 16
        gt_k[pl.ds(cgx + a * 16, 16)] = jnp.where(lanepos < cex, Tx, _U(0))
        gt_i[pl.ds(cgx + a * 16, 16)] = eq_i[pl.ds(a * 16, 16)]

      return (cgx, cex)

    maxv = jnp.zeros((16,), jnp.uint32)
    for vi in range(GT_BUF // 16):
      maxv = jnp.where(iota == vi, jnp.max(gt_k[pl.ds(vi * 16, 16)]), maxv)

    def select(j, carry):
      maxv, vvec, ivec = carry
      m = jnp.max(maxv)
      vb = _scalar(plsc.all_reduce_ffs(maxv == m))
      kv = gt_k[pl.ds(vb * 16, 16)]
      lane = _scalar(plsc.all_reduce_ffs(kv == m))
      pos = vb * 16 + lane
      isel = _scalar(plsc.load_gather(gt_i, [jnp.full((16,), pos, jnp.int32)]))
      kv2 = jnp.where(iota == lane, _U(0), kv)
      gt_k[pl.ds(vb * 16, 16)] = kv2
      maxv = jnp.where(iota == vb, jnp.max(kv2), maxv)

      bits = m ^ ((~m >> _U(31)) | _U(0x80000000))
      val = lax.bitcast_convert_type(bits, jnp.float32)
      fidx = isel.astype(jnp.float32)
      l = j & 15
      base = j - l
      vvec = jnp.where(iota == l, val, vvec)
      ivec = jnp.where(iota == l, fidx, ivec)
      outst[pl.ds(base, 16)] = ivec
      outst[pl.ds(K + base, 16)] = vvec
      return (maxv, vvec, ivec)

    zf = jnp.zeros((16,), jnp.float32)
    lax.fori_loop(0, K, select, (maxv, zf, zf))

    pltpu.sync_copy(outst, out_hbm.at[row])

  r0 = wid * 4
  pltpu.async_copy(in_hbm.at[r0], row_a, sem_a)

  def pair(p, _):
    ra = r0 + 2 * p
    pltpu.make_async_copy(in_hbm.at[ra], row_a, sem_a).wait()
    pltpu.async_copy(in_hbm.at[ra + 1], row_b, sem_b)
    process(row_a, ra)
    pltpu.make_async_copy(in_hbm.at[ra + 1], row_b, sem_b).wait()
    nxt = jnp.minimum(ra + 2, ROWS - 1)
    pltpu.async_copy(in_hbm.at[nxt], row_a, sem_a)
    process(row_b, ra + 1)
    return 0

  lax.fori_loop(0, 2, pair, 0)
  last = jnp.minimum(r0 + 4, ROWS - 1)
  pltpu.make_async_copy(in_hbm.at[last], row_a, sem_a).wait()


_sc_topk = functools.partial(
    pl.kernel,
    out_type=jax.ShapeDtypeStruct((ROWS, 2 * K), jnp.float32),
    mesh=plsc.VectorSubcoreMesh(core_axis_name="c", subcore_axis_name="s"),
    compiler_params=pltpu.CompilerParams(needs_layout_passes=False),
    scratch_types=[
        pltpu.VMEM((N,), jnp.float32),
        pltpu.VMEM((N,), jnp.float32),
        pltpu.VMEM((4352,), jnp.int32),
        pltpu.VMEM((GT_BUF,), jnp.uint32),
        pltpu.VMEM((GT_BUF,), jnp.int32),
        pltpu.VMEM((EQ_BUF,), jnp.int32),
        pltpu.VMEM((2 * K,), jnp.float32),
        pltpu.SemaphoreType.DMA,
        pltpu.SemaphoreType.DMA,
    ],
)(_sc_body)


def _norm_body(x_ref, o_ref):
  x = x_ref[...]
  val = x[:, K:]
  sc = jnp.sum(jnp.abs(val)) + 1e-6
  o_ref[:, :K] = x[:, :K]
  o_ref[:, K:] = val / sc


_norm = pl.pallas_call(
    _norm_body,
    out_shape=jax.ShapeDtypeStruct((ROWS, 2 * K), jnp.float32),
)


def kernel(inputs):
  return _norm(_sc_topk(inputs))


# speedup vs baseline: 1.2012x; 1.2012x over previous
import functools

import jax
import jax.numpy as jnp
from jax import lax
from jax.experimental import pallas as pl
from jax.experimental.pallas import tpu as pltpu
from jax.experimental.pallas import tpu_sc as plsc

K = 64
N = 32768
ROWS = 128
NVEC = N // 16
GT_BUF = 256
EQ_BUF = 80
GT_CAP = 240
HIST_EXIT = 96
KSAMP = 32

_U = jnp.uint32


def _keys(v):
  kb = plsc.bitcast(v, jnp.uint32)
  sign = kb >> _U(31)
  flip = (_U(0) - sign) | _U(0x80000000)
  return kb ^ flip


def _scalar(x):
  return x[0]


def _sc_body(in_hbm, out_hbm, row_a, row_b, hist_v, gt_k, gt_i, eq_i, outst,
             sem_a, sem_b):
  c = lax.axis_index("c")
  s = lax.axis_index("s")
  wid = c * 16 + s
  iota = lax.iota(jnp.int32, 16)
  ones = jnp.ones((16,), jnp.int32)
  zeros_u = jnp.zeros((16,), jnp.uint32)
  zeros_i = jnp.zeros((16,), jnp.int32)

  def clear_hist(nwords=4096):
    @plsc.parallel_loop(0, nwords, step=16, unroll=8)
    def _hc(i):
      hist_v[pl.ds(i, 16)] = zeros_i

  def clear_gt():
    for ci in range(GT_BUF // 16):
      gt_k[pl.ds(ci * 16, 16)] = zeros_u

  def scan_hist(base, nb_vec, kneed, acc0):
    static = isinstance(nb_vec, int)
    init = (acc0, jnp.int32(0), acc0, jnp.bool_(False))

    @pl.loop(0, nb_vec, init_carry=init, unroll=8 if static and nb_vec >= 8
             else None)
    def scan_res(i, sc):
      acc, vvi, vacc, found = sc
      vi = nb_vec - 1 - i
      cum = jnp.cumsum(hist_v[pl.ds(base + vi * 16, 16)])
      acc2 = acc + cum[15]
      crossed = jnp.logical_and(acc2 >= kneed, jnp.logical_not(found))
      vvi = jnp.where(crossed, vi, vvi)
      vacc = jnp.where(crossed, acc, vacc)
      return (acc2, vvi, vacc, found | crossed)

    _, vvi, vacc, _ = scan_res
    vec = hist_v[pl.ds(base + vvi * 16, 16)]
    cum = jnp.cumsum(vec)
    suf = vacc + cum[15] - cum + vec
    m = suf >= kneed
    blane = jnp.max(jnp.where(m, iota, -1))
    sel = iota == blane
    SB = jnp.max(jnp.where(sel, suf, 0))
    histB = jnp.max(jnp.where(sel, vec, 0))
    return vvi * 16 + blane, SB, histB

  def collect_gt(row_v, T):
    @plsc.parallel_loop(0, N, step=16, unroll=8,
                        carry=jnp.zeros((16,), jnp.int32))
    def cnts(i, cgv):
      ku = _keys(row_v[pl.ds(i, 16)])
      mgt = ku > T
      cs = jnp.minimum(_scalar(cgv), GT_CAP)
      plsc.store_compressed(gt_k.at[pl.ds(cs, 16)], ku, mask=mgt)
      plsc.store_compressed(gt_i.at[pl.ds(cs, 16)], iota + i, mask=mgt)
      return cgv + plsc.all_reduce_population_count(mgt)

    return _scalar(cnts)

  def process(row_v, row):
    clear_gt()
    clear_hist(4352)

    @plsc.parallel_loop(0, N, step=64, unroll=8)
    def _samp(i):
      ku = _keys(row_v[pl.ds(i, 16)])
      b = (ku >> _U(20)).astype(jnp.int32)
      plsc.addupdate_scatter(hist_v, [b], ones)
      plsc.addupdate_scatter(hist_v, [4096 + (b >> 4)], ones)

    ks = jnp.int32(KSAMP)
    G, SG, hG = scan_hist(4096, 16, ks, jnp.int32(0))
    Br, _, _ = scan_hist(G * 16, 1, ks, SG - hG)
    T = (G * 16 + Br).astype(jnp.uint32) << _U(20)

    ct = collect_gt(row_v, T)

    fb = jnp.logical_or(ct < K, ct > GT_CAP)

    @pl.loop(0, jnp.where(fb, 1, 0), init_carry=(ct, jnp.int32(0)))
    def cg_ce(_t, _c):
      clear_hist()

      @plsc.parallel_loop(0, N, step=16, unroll=8)
      def _full(i):
        ku = _keys(row_v[pl.ds(i, 16)])
        b = (ku >> _U(20)).astype(jnp.int32)
        plsc.addupdate_scatter(hist_v, [b], ones)

      B, SB, histB = scan_hist(0, 256, jnp.int32(K), jnp.int32(0))
      T0 = B.astype(jnp.uint32) << _U(20)

      refine = histB > HIST_EXIT
      init = (_U(0xFFF00000), T0, jnp.int32(K) - (SB - histB), T0)

      @pl.loop(0, jnp.where(refine, 2, 0), init_carry=init)
      def ref_res(l, cr):
        pmask, pval, kneed, _ = cr
        is1 = l == 0
        shift = jnp.where(is1, _U(8), _U(0))
        bmask = jnp.where(is1, _U(0xFFF), _U(0xFF))
        nb_vec = jnp.where(is1, 256, 16)

        @pl.loop(0, nb_vec)
        def _c(ci):
          hist_v[pl.ds(ci * 16, 16)] = zeros_i

        def hp(i, _):
          ku = _keys(row_v[pl.ds(i * 16, 16)])
          part = (ku & pmask) == pval
          b = ((ku >> shift) & bmask).astype(jnp.int32)
          plsc.addupdate_scatter(hist_v, [b], ones, mask=part)
          return 0
        lax.fori_loop(0, NVEC, hp, 0)

        B2, SB2, histB2 = scan_hist(0, nb_vec, kneed, jnp.int32(0))
        T2 = pval | (B2.astype(jnp.uint32) << shift)
        return (pmask | (bmask << shift), T2, kneed - (SB2 - histB2), T2)

      Tx = ref_res[3]
      clear_gt()
      cgx = collect_gt(row_v, Tx)

      @pl.loop(0, jnp.where(cgx < K, 1, 0), init_carry=jnp.int32(0))
      def cex(_t2, _ce):
        def eqp(i, cev):
          ku = _keys(row_v[pl.ds(i * 16, 16)])
          meq = jnp.logical_and(ku == Tx, cev < K)
          plsc.store_compressed(eq_i.at[pl.ds(_scalar(cev), 16)],
                                iota + i * 16, mask=meq)
          return cev + plsc.all_reduce_population_count(meq)
        return _scalar(lax.fori_loop(0, NVEC, eqp, zeros_i))

      for a in range(EQ_BUF // 16):
        lanepos = iota + a * 16
        gt_k[pl.ds(cgx + a * 16, 16)] = jnp.where(lanepos < cex, Tx, _U(0))
        gt_i[pl.ds(cgx + a * 16, 16)] = eq_i[pl.ds(a * 16, 16)]

      return (cgx, cex)

    maxv = jnp.zeros((16,), jnp.uint32)
    for vi in range(GT_BUF // 16):
      maxv = jnp.where(iota == vi, jnp.max(gt_k[pl.ds(vi * 16, 16)]), maxv)

    def select(j, carry):
      maxv, vvec, ivec = carry
      m = jnp.max(maxv)
      vb = _scalar(plsc.all_reduce_ffs(maxv == m))
      kv = gt_k[pl.ds(vb * 16, 16)]
      lane = _scalar(plsc.all_reduce_ffs(kv == m))
      pos = vb * 16 + lane
      isel = _scalar(plsc.load_gather(gt_i, [jnp.full((16,), pos, jnp.int32)]))
      kv2 = jnp.where(iota == lane, _U(0), kv)
      gt_k[pl.ds(vb * 16, 16)] = kv2
      maxv = jnp.where(iota == vb, jnp.max(kv2), maxv)

      bits = m ^ ((~m >> _U(31)) | _U(0x80000000))
      val = lax.bitcast_convert_type(bits, jnp.float32)
      fidx = isel.astype(jnp.float32)
      l = j & 15
      base = j - l
      vvec = jnp.where(iota == l, val, vvec)
      ivec = jnp.where(iota == l, fidx, ivec)
      outst[pl.ds(base, 16)] = ivec
      outst[pl.ds(K + base, 16)] = vvec
      return (maxv, vvec, ivec)

    zf = jnp.zeros((16,), jnp.float32)
    lax.fori_loop(0, K, select, (maxv, zf, zf))

    pltpu.sync_copy(outst, out_hbm.at[row])

  r0 = wid * 4
  pltpu.async_copy(in_hbm.at[r0], row_a, sem_a)

  def pair(p, _):
    ra = r0 + 2 * p
    pltpu.make_async_copy(in_hbm.at[ra], row_a, sem_a).wait()
    pltpu.async_copy(in_hbm.at[ra + 1], row_b, sem_b)
    process(row_a, ra)
    pltpu.make_async_copy(in_hbm.at[ra + 1], row_b, sem_b).wait()
    nxt = jnp.minimum(ra + 2, ROWS - 1)
    pltpu.async_copy(in_hbm.at[nxt], row_a, sem_a)
    process(row_b, ra + 1)
    return 0

  lax.fori_loop(0, 2, pair, 0)
  last = jnp.minimum(r0 + 4, ROWS - 1)
  pltpu.make_async_copy(in_hbm.at[last], row_a, sem_a).wait()


_sc_topk = functools.partial(
    pl.kernel,
    out_type=jax.ShapeDtypeStruct((ROWS, 2 * K), jnp.float32),
    mesh=plsc.VectorSubcoreMesh(core_axis_name="c", subcore_axis_name="s"),
    compiler_params=pltpu.CompilerParams(needs_layout_passes=False),
    scratch_types=[
        pltpu.VMEM((N,), jnp.float32),
        pltpu.VMEM((N,), jnp.float32),
        pltpu.VMEM((4352,), jnp.int32),
        pltpu.VMEM((GT_BUF,), jnp.uint32),
        pltpu.VMEM((GT_BUF,), jnp.int32),
        pltpu.VMEM((EQ_BUF,), jnp.int32),
        pltpu.VMEM((2 * K,), jnp.float32),
        pltpu.SemaphoreType.DMA,
        pltpu.SemaphoreType.DMA,
    ],
)(_sc_body)


def _norm_body(x_ref, o_ref):
  x = x_ref[...]
  val = x[:, K:]
  sc = jnp.sum(jnp.abs(val)) + 1e-6
  o_ref[:, :K] = x[:, :K]
  o_ref[:, K:] = val / sc


_norm = pl.pallas_call(
    _norm_body,
    out_shape=jax.ShapeDtypeStruct((ROWS, 2 * K), jnp.float32),
)


def kernel(inputs):
  return _norm(_sc_topk(inputs))
